# Initial kernel scaffold; baseline (speedup 1.0000x reference)
import jax
import jax.numpy as jnp
from jax.experimental import pallas as pl


def kernel(x, edge_index, W1, a_src1, a_dst1, b1, W2, a_src2, a_dst2, b2):
    raise NotImplementedError("write your pallas kernel here")


# speedup vs baseline: 35.6994x; 35.6994x over previous
import functools

import jax
import jax.numpy as jnp
from jax import lax
from jax.experimental import pallas as pl
from jax.experimental.pallas import tpu as pltpu
from jax.experimental.pallas import tpu_sc as plsc

AG = 5
NN = 10000
EE = 160000
DD = 256
NH = 8
CH = 32
LL = 16
NS = 16
NP = 10240
EPS = 1e-16

EPT = EE // NS
CHK = 80
NCHK = EPT // CHK
NSL = NP // NS


def _al(x):
    return pl.multiple_of(x, 8)


def _stage_a(x, W1, Asp, Adp):
    def body(x_ref, w_ref, as_ref, ad_ref, h_ref, s_ref, d_ref):
        h = jnp.dot(x_ref[0], w_ref[...], preferred_element_type=jnp.float32)
        h_ref[0] = h
        s_ref[0] = jnp.dot(h, as_ref[...], preferred_element_type=jnp.float32)
        d_ref[0] = jnp.dot(h, ad_ref[...], preferred_element_type=jnp.float32)

    blk = 1000
    grid = (AG, NN // blk)
    return pl.pallas_call(
        body,
        grid=grid,
        in_specs=[
            pl.BlockSpec((1, blk, DD), lambda g, i: (g, i, 0)),
            pl.BlockSpec((DD, DD), lambda g, i: (0, 0)),
            pl.BlockSpec((DD, LL), lambda g, i: (0, 0)),
            pl.BlockSpec((DD, LL), lambda g, i: (0, 0)),
        ],
        out_specs=[
            pl.BlockSpec((1, blk, DD), lambda g, i: (g, i, 0)),
            pl.BlockSpec((1, blk, LL), lambda g, i: (g, i, 0)),
            pl.BlockSpec((1, blk, LL), lambda g, i: (g, i, 0)),
        ],
        out_shape=[
            jax.ShapeDtypeStruct((AG, NN, DD), jnp.float32),
            jax.ShapeDtypeStruct((AG, NN, LL), jnp.float32),
            jax.ShapeDtypeStruct((AG, NN, LL), jnp.float32),
        ],
    )(x, W1, Asp, Adp)


def _stage_b(as1p, ad1p, srcg, dstg):
    mesh = plsc.VectorSubcoreMesh(core_axis_name="c", subcore_axis_name="s")

    @functools.partial(
        pl.kernel,
        out_type=[
            jax.ShapeDtypeStruct((AG * EE, LL), jnp.float32),
            jax.ShapeDtypeStruct((AG * NP, LL), jnp.float32),
        ],
        mesh=mesh,
        compiler_params=pltpu.CompilerParams(use_tc_tiling_on_sc=False, needs_layout_passes=False),
        scratch_types=[
            pltpu.VMEM_SHARED((NP, LL), jnp.float32),
            pltpu.VMEM((CHK,), jnp.int32),
            pltpu.VMEM((CHK,), jnp.int32),
            pltpu.VMEM((CHK,), jnp.int32),
            pltpu.VMEM((CHK, LL), jnp.float32),
            pltpu.VMEM((CHK, LL), jnp.float32),
            pltpu.VMEM((CHK, LL), jnp.float32),
            pltpu.VMEM((NSL, LL), jnp.float32),
            pltpu.VMEM((NSL, LL), jnp.float32),
            pltpu.VMEM((NSL, LL), jnp.float32),
            pltpu.SemaphoreType.DMA,
            pltpu.SemaphoreType.DMA,
        ],
    )
    def kb(as_hbm, ad_hbm, src_hbm, dst_hbm, w_hbm, rden_hbm,
           den_sp, sv, dv, dl, asr, adr, wr, zb, dc, rb, sem1, sem2):
        core = lax.axis_index("c")
        sid = lax.axis_index("s")

        @pl.loop(0, NSL)
        def _zb(i):
            zb[i, :] = jnp.zeros((LL,), jnp.float32)

        for g in range(AG):
            @pl.when(core == g % 2)
            def _g():
                pltpu.sync_copy(zb, den_sp.at[pl.ds(_al(sid * NSL), NSL)])
                plsc.subcore_barrier()

                @pl.loop(0, NCHK)
                def _chunk(ci):
                    base = _al(g * EE + sid * EPT + ci * CHK)
                    pltpu.sync_copy(src_hbm.at[pl.ds(base, CHK)], sv)
                    pltpu.sync_copy(dst_hbm.at[pl.ds(base, CHK)], dv)
                    cp1 = pltpu.async_copy(as_hbm.at[sv], asr, sem1)
                    cp2 = pltpu.async_copy(ad_hbm.at[dv], adr, sem2)
                    for k in range(CHK // LL):
                        d16 = dv[pl.ds(k * LL, LL)]
                        dl[pl.ds(k * LL, LL)] = d16 - g * NP
                    cp1.wait()
                    cp2.wait()

                    @pl.loop(0, CHK)
                    def _e(i):
                        t = asr[i, :] + adr[i, :]
                        wr[i, :] = jnp.exp(jnp.maximum(t, 0.2 * t))

                    pltpu.sync_copy(wr, den_sp.at[dl], add=True)
                    pltpu.sync_copy(wr, w_hbm.at[pl.ds(base, CHK), :])

                plsc.subcore_barrier()
                pltpu.sync_copy(den_sp.at[pl.ds(_al(sid * NSL), NSL)], dc)

                @pl.loop(0, NSL)
                def _r(i):
                    rb[i, :] = 1.0 / (dc[i, :] + EPS)

                pltpu.sync_copy(
                    rb, rden_hbm.at[pl.ds(_al(g * NP + sid * NSL), NSL), :])

    return kb(as1p, ad1p, srcg, dstg)


def _stage_c(h1g, w, rden, srcg, dstg):
    mesh = plsc.VectorSubcoreMesh(core_axis_name="c", subcore_axis_name="s")
    ZR = 160

    @functools.partial(
        pl.kernel,
        out_type=jax.ShapeDtypeStruct((AG, 2, NP, 128), jnp.float32),
        mesh=mesh,
        compiler_params=pltpu.CompilerParams(use_tc_tiling_on_sc=False, needs_layout_passes=False),
        scratch_types=[
            pltpu.VMEM_SHARED((NP, 128), jnp.float32),
            pltpu.VMEM((CHK,), jnp.int32),
            pltpu.VMEM((CHK,), jnp.int32),
            pltpu.VMEM((CHK,), jnp.int32),
            pltpu.VMEM((CHK,), jnp.int32),
            pltpu.VMEM((CHK, 128), jnp.float32),
            pltpu.VMEM((CHK, LL), jnp.float32),
            pltpu.VMEM((CHK, LL), jnp.float32),
            pltpu.VMEM((ZR, 128), jnp.float32),
            pltpu.SemaphoreType.DMA,
            pltpu.SemaphoreType.DMA,
        ],
    )
    def kc(h_hbm, w_hbm, rd_hbm, src_hbm, dst_hbm, out_hbm,
           acc_sp, sv, dv, dl, gi, hr, wr, rr, zb, sem1, sem2):
        core = lax.axis_index("c")
        sid = lax.axis_index("s")

        @pl.loop(0, ZR)
        def _zb(i):
            for v in range(8):
                zb[i, pl.ds(v * LL, LL)] = jnp.zeros((LL,), jnp.float32)

        for g in range(AG):
            for grp in range(2):
                @pl.when(core == grp)
                def _t():
                    for j in range(NSL // ZR):
                        pltpu.sync_copy(
                            zb, acc_sp.at[pl.ds(_al(sid * NSL + j * ZR), ZR)])
                    plsc.subcore_barrier()

                    @pl.loop(0, NCHK)
                    def _chunk(ci):
                        base = _al(g * EE + sid * EPT + ci * CHK)
                        pltpu.sync_copy(src_hbm.at[pl.ds(base, CHK)], sv)
                        pltpu.sync_copy(dst_hbm.at[pl.ds(base, CHK)], dv)
                        for k in range(CHK // LL):
                            s16 = sv[pl.ds(k * LL, LL)]
                            gi[pl.ds(k * LL, LL)] = s16 * 2 + grp
                            d16 = dv[pl.ds(k * LL, LL)]
                            dl[pl.ds(k * LL, LL)] = d16 - g * NP
                        cp1 = pltpu.async_copy(h_hbm.at[gi], hr, sem1)
                        cp2 = pltpu.async_copy(rd_hbm.at[dv], rr, sem2)
                        pltpu.sync_copy(w_hbm.at[pl.ds(base, CHK), :], wr)
                        cp1.wait()
                        cp2.wait()

                        @pl.loop(0, CHK)
                        def _e(i):
                            arow = wr[i, :] * rr[i, :]
                            for j in range(4):
                                a = arow[grp * 4 + j]
                                for v in range(2):
                                    sl = pl.ds(j * 32 + v * LL, LL)
                                    hr[i, sl] = hr[i, sl] * a

                        pltpu.sync_copy(hr, acc_sp.at[dl], add=True)

                    plsc.subcore_barrier()
                    pltpu.sync_copy(
                        acc_sp.at[pl.ds(_al(sid * NSL), NSL)],
                        out_hbm.at[g, grp, pl.ds(_al(sid * NSL), NSL), :])

    return kc(h1g, w, rden, srcg, dstg)


def _stage_d(acc, b1p, W2p):
    def body(a_ref, b_ref, w_ref, o_ref):
        h0 = jnp.maximum(a_ref[0, 0] + b_ref[0], 0.0)
        h1 = jnp.maximum(a_ref[0, 1] + b_ref[1], 0.0)
        o_ref[0] = (jnp.dot(h0, w_ref[0], preferred_element_type=jnp.float32)
                    + jnp.dot(h1, w_ref[1],
                              preferred_element_type=jnp.float32))

    blk = 1024
    return pl.pallas_call(
        body,
        grid=(AG, NP // blk),
        in_specs=[
            pl.BlockSpec((1, 2, blk, 128), lambda g, i: (g, 0, i, 0)),
            pl.BlockSpec((2, 128), lambda g, i: (0, 0)),
            pl.BlockSpec((2, 128, 1), lambda g, i: (0, 0, 0)),
        ],
        out_specs=pl.BlockSpec((1, blk, 1), lambda g, i: (g, i, 0)),
        out_shape=jax.ShapeDtypeStruct((AG, NP, 1), jnp.float32),
    )(acc, b1p, W2p)


def _stage_e(s2, d2, h2, srcg, dstg):
    mesh = plsc.VectorSubcoreMesh(core_axis_name="c", subcore_axis_name="s")
    NR = NP // LL
    RS = NR // NS

    @functools.partial(
        pl.kernel,
        out_type=[
            jax.ShapeDtypeStruct((AG, NR, LL), jnp.float32),
            jax.ShapeDtypeStruct((AG, NR, LL), jnp.float32),
            jax.ShapeDtypeStruct((AG, NP), jnp.float32),
        ],
        mesh=mesh,
        compiler_params=pltpu.CompilerParams(use_tc_tiling_on_sc=False, needs_layout_passes=False),
        scratch_types=[
            pltpu.VMEM_SHARED((NR, LL), jnp.float32),
            pltpu.VMEM_SHARED((NR, LL), jnp.float32),
            pltpu.VMEM_SHARED((NR, LL), jnp.float32),
            pltpu.VMEM((NP,), jnp.float32),
            pltpu.VMEM((NP,), jnp.float32),
            pltpu.VMEM((NP,), jnp.float32),
            pltpu.VMEM((NR, LL), jnp.float32),
            pltpu.VMEM((NR, LL), jnp.float32),
            pltpu.VMEM((NR, LL), jnp.float32),
            pltpu.VMEM((NR, LL), jnp.float32),
            pltpu.VMEM((NR,), jnp.int32),
            pltpu.VMEM((CHK,), jnp.int32),
            pltpu.VMEM((CHK,), jnp.int32),
            pltpu.VMEM((NP,), jnp.float32),
            pltpu.VMEM((RS, LL), jnp.float32),
            pltpu.VMEM((RS, LL), jnp.float32),
            pltpu.VMEM((NSL,), jnp.float32),
            pltpu.VMEM((NSL,), jnp.int32),
            pltpu.VMEM((RS, LL), jnp.float32),
            pltpu.VMEM((RS, LL), jnp.float32),
        ],
    )
    def ke(s2_hbm, d2_hbm, h2_hbm, src_hbm, dst_hbm,
           out2_hbm, att_hbm, ee2_hbm,
           den_sp, num_sp, cnt_sp,
           s2t, d2t, h2t, denp, nump, cntp, denf, idn,
           sv, dv, eef, nsl, csl, esl, dnv, ob, ab):
        core = lax.axis_index("c")
        sid = lax.axis_index("s")

        @pl.loop(0, RS)
        def _idn(k):
            iota = lax.broadcasted_iota(jnp.int32, (LL,), 0)
            idn[pl.ds(k * LL, LL)] = iota + k * LL

        for g in range(AG):
            @pl.when(core == g % 2)
            def _g():
                @pl.loop(0, NR)
                def _z(i):
                    z = jnp.zeros((LL,), jnp.float32)
                    denp[i, :] = z
                    nump[i, :] = z
                    cntp[i, :] = z

                sl = pl.ds(_al(sid * RS), RS)
                pltpu.sync_copy(denp.at[pl.ds(0, RS)], den_sp.at[sl])
                pltpu.sync_copy(denp.at[pl.ds(0, RS)], num_sp.at[sl])
                pltpu.sync_copy(denp.at[pl.ds(0, RS)], cnt_sp.at[sl])
                pltpu.sync_copy(s2_hbm.at[g], s2t)
                pltpu.sync_copy(d2_hbm.at[g], d2t)
                pltpu.sync_copy(h2_hbm.at[g], h2t)
                plsc.subcore_barrier()

                @pl.loop(0, NCHK)
                def _chunk(ci):
                    base = _al(g * EE + sid * EPT + ci * CHK)
                    pltpu.sync_copy(src_hbm.at[pl.ds(base, CHK)], sv)
                    pltpu.sync_copy(dst_hbm.at[pl.ds(base, CHK)], dv)
                    ones = jnp.ones((LL,), jnp.float32)
                    for k in range(CHK // LL):
                        s16 = sv[pl.ds(k * LL, LL)] - g * NP
                        d16 = dv[pl.ds(k * LL, LL)] - g * NP
                        t = (plsc.load_gather(s2t, [s16])
                             + plsc.load_gather(d2t, [d16]))
                        w2 = jnp.exp(jnp.maximum(t, 0.2 * t))
                        eef[pl.ds(ci * CHK + k * LL, LL)] = w2
                        h2v = plsc.load_gather(h2t, [s16])
                        row = lax.shift_right_logical(d16, 4)
                        col = jnp.bitwise_and(d16, LL - 1)
                        plsc.addupdate_scatter(denp, [row, col], w2)
                        plsc.addupdate_scatter(nump, [row, col], w2 * h2v)
                        plsc.addupdate_scatter(cntp, [row, col], ones)

                @pl.when(sid == 0)
                def _ee():
                    pltpu.sync_copy(eef.at[pl.ds(0, EPT)],
                                    ee2_hbm.at[g, pl.ds(0, EPT)])

                pltpu.sync_copy(denp, den_sp.at[idn], add=True)
                pltpu.sync_copy(nump, num_sp.at[idn], add=True)
                pltpu.sync_copy(cntp, cnt_sp.at[idn], add=True)
                plsc.subcore_barrier()

                pltpu.sync_copy(den_sp, denf)
                pltpu.sync_copy(num_sp.at[sl], nsl)
                pltpu.sync_copy(cnt_sp.at[sl], csl)
                pltpu.sync_copy(ee2_hbm.at[g, pl.ds(_al(sid * NSL), NSL)],
                                esl)
                pltpu.sync_copy(
                    dst_hbm.at[pl.ds(_al(g * EE + sid * NSL), NSL)], dnv)

                @pl.loop(0, RS)
                def _o(r):
                    dr = denf[sid * RS + r, :]
                    ob[r, :] = nsl[r, :] / (dr + EPS)

                pltpu.sync_copy(ob, out2_hbm.at[g, sl, :])

                @pl.loop(0, RS)
                def _a(k):
                    d16 = dnv[pl.ds(k * LL, LL)] - g * NP
                    row = lax.shift_right_logical(d16, 4)
                    col = jnp.bitwise_and(d16, LL - 1)
                    denv = plsc.load_gather(denf, [row, col])
                    e16 = esl[pl.ds(k * LL, LL)]
                    ab[k, :] = csl[k, :] * e16 / (denv + EPS)

                pltpu.sync_copy(ab, att_hbm.at[g, sl, :])

    return ke(s2, d2, h2, srcg, dstg)


def _stage_f(out2, b2p):
    def body(x_ref, b_ref, o_ref):
        s = jnp.sum(x_ref[...]) * (1.0 / NN) + b_ref[0, 0]
        o_ref[...] = jnp.full((1, 1, 128), s, jnp.float32)

    return pl.pallas_call(
        body,
        grid=(AG,),
        in_specs=[
            pl.BlockSpec((1, NP // 128, 128), lambda g: (g, 0, 0)),
            pl.BlockSpec((1, 1), lambda g: (0, 0)),
        ],
        out_specs=pl.BlockSpec((1, 1, 128), lambda g: (g, 0, 0)),
        out_shape=jax.ShapeDtypeStruct((AG, 1, 128), jnp.float32),
    )(out2, b2p)


def kernel(x, edge_index, W1, a_src1, a_dst1, b1, W2, a_src2, a_dst2, b2):
    eye = jnp.eye(NH, dtype=jnp.float32)
    Asp = jnp.pad((a_src1[:, :, None] * eye[:, None, :]).reshape(DD, NH),
                  ((0, 0), (0, NH)))
    Adp = jnp.pad((a_dst1[:, :, None] * eye[:, None, :]).reshape(DD, NH),
                  ((0, 0), (0, NH)))
    off = (jnp.arange(AG, dtype=jnp.int32) * NP)[:, None]
    srcg = (edge_index[:, 0, :] + off).reshape(AG * EE)
    dstg = (edge_index[:, 1, :] + off).reshape(AG * EE)

    h1, as1p, ad1p = _stage_a(x, W1, Asp, Adp)
    padn = ((0, 0), (0, NP - NN), (0, 0))
    as1p = jnp.pad(as1p, padn).reshape(AG * NP, LL)
    ad1p = jnp.pad(ad1p, padn).reshape(AG * NP, LL)
    h1g = jnp.pad(h1, padn).reshape(AG * NP * 2, 128)
    w, rden = _stage_b(as1p, ad1p, srcg, dstg)
    acc = _stage_c(h1g, w, rden, srcg, dstg)
    h2 = _stage_d(acc, b1.reshape(2, 128), W2.reshape(2, 128, 1))
    h2f = h2[:, :, 0]
    s2 = h2f * a_src2[0, 0]
    d2 = h2f * a_dst2[0, 0]
    out2, att, _ = _stage_e(s2, d2, h2f, srcg, dstg)
    probas = _stage_f(out2.reshape(AG, NP // 128, 128), b2.reshape(1, 1))
    proba_vec = probas[:, 0, 0]
    best = jnp.argmax(proba_vec)
    attention = jnp.take(att.reshape(AG, NP)[:, :NN], best, axis=0)
    return proba_vec, attention


# speedup vs baseline: 43.6118x; 1.2216x over previous
import functools

import jax
import jax.numpy as jnp
from jax import lax
from jax.experimental import pallas as pl
from jax.experimental.pallas import tpu as pltpu
from jax.experimental.pallas import tpu_sc as plsc

AG = 5
NN = 10000
EE = 160000
DD = 256
NH = 8
CH = 32
LL = 16
NS = 16
NP = 10240
EPS = 1e-16

EPT = EE // NS
CHK = 80
NCHK = EPT // CHK
NSL = NP // NS


def _al(x):
    return pl.multiple_of(x, 8)


def _stage_a(x, W1, Asp, Adp):
    def body(x_ref, w_ref, as_ref, ad_ref, h_ref, s_ref, d_ref):
        h = jnp.dot(x_ref[0], w_ref[...], preferred_element_type=jnp.float32)
        h_ref[0] = h
        s_ref[0] = jnp.dot(h, as_ref[...], preferred_element_type=jnp.float32)
        d_ref[0] = jnp.dot(h, ad_ref[...], preferred_element_type=jnp.float32)

    blk = 1000
    grid = (AG, NN // blk)
    return pl.pallas_call(
        body,
        grid=grid,
        in_specs=[
            pl.BlockSpec((1, blk, DD), lambda g, i: (g, i, 0)),
            pl.BlockSpec((DD, DD), lambda g, i: (0, 0)),
            pl.BlockSpec((DD, LL), lambda g, i: (0, 0)),
            pl.BlockSpec((DD, LL), lambda g, i: (0, 0)),
        ],
        out_specs=[
            pl.BlockSpec((1, blk, DD), lambda g, i: (g, i, 0)),
            pl.BlockSpec((1, blk, LL), lambda g, i: (g, i, 0)),
            pl.BlockSpec((1, blk, LL), lambda g, i: (g, i, 0)),
        ],
        out_shape=[
            jax.ShapeDtypeStruct((AG, NN, DD), jnp.float32),
            jax.ShapeDtypeStruct((AG, NN, LL), jnp.float32),
            jax.ShapeDtypeStruct((AG, NN, LL), jnp.float32),
        ],
    )(x, W1, Asp, Adp)


def _stage_b(as1p, ad1p, srcg, dstg):
    mesh = plsc.VectorSubcoreMesh(core_axis_name="c", subcore_axis_name="s")

    @functools.partial(
        pl.kernel,
        out_type=[
            jax.ShapeDtypeStruct((AG * EE, LL), jnp.float32),
            jax.ShapeDtypeStruct((AG * NP, LL), jnp.float32),
        ],
        mesh=mesh,
        compiler_params=pltpu.CompilerParams(use_tc_tiling_on_sc=False, needs_layout_passes=False),
        scratch_types=[
            pltpu.VMEM_SHARED((NP, LL), jnp.float32),
            pltpu.VMEM((CHK,), jnp.int32),
            pltpu.VMEM((CHK,), jnp.int32),
            pltpu.VMEM((CHK,), jnp.int32),
            pltpu.VMEM((CHK, LL), jnp.float32),
            pltpu.VMEM((CHK, LL), jnp.float32),
            pltpu.VMEM((CHK, LL), jnp.float32),
            pltpu.VMEM((NSL, LL), jnp.float32),
            pltpu.VMEM((NSL, LL), jnp.float32),
            pltpu.VMEM((NSL, LL), jnp.float32),
            pltpu.SemaphoreType.DMA,
            pltpu.SemaphoreType.DMA,
        ],
    )
    def kb(as_hbm, ad_hbm, src_hbm, dst_hbm, w_hbm, rden_hbm,
           den_sp, sv, dv, dl, asr, adr, wr, zb, dc, rb, sem1, sem2):
        core = lax.axis_index("c")
        sid = lax.axis_index("s")

        @pl.loop(0, NSL)
        def _zb(i):
            zb[i, :] = jnp.zeros((LL,), jnp.float32)

        for g in range(AG):
            @pl.when(core == g % 2)
            def _g():
                pltpu.sync_copy(zb, den_sp.at[pl.ds(_al(sid * NSL), NSL)])
                plsc.subcore_barrier()

                @pl.loop(0, NCHK)
                def _chunk(ci):
                    base = _al(g * EE + sid * EPT + ci * CHK)
                    pltpu.sync_copy(src_hbm.at[pl.ds(base, CHK)], sv)
                    pltpu.sync_copy(dst_hbm.at[pl.ds(base, CHK)], dv)
                    cp1 = pltpu.async_copy(as_hbm.at[sv], asr, sem1)
                    cp2 = pltpu.async_copy(ad_hbm.at[dv], adr, sem2)
                    for k in range(CHK // LL):
                        d16 = dv[pl.ds(k * LL, LL)]
                        dl[pl.ds(k * LL, LL)] = d16 - g * NP
                    cp1.wait()
                    cp2.wait()

                    @pl.loop(0, CHK)
                    def _e(i):
                        t = asr[i, :] + adr[i, :]
                        wr[i, :] = jnp.exp(jnp.maximum(t, 0.2 * t))

                    pltpu.sync_copy(wr, den_sp.at[dl], add=True)
                    pltpu.sync_copy(wr, w_hbm.at[pl.ds(base, CHK), :])

                plsc.subcore_barrier()
                pltpu.sync_copy(den_sp.at[pl.ds(_al(sid * NSL), NSL)], dc)

                @pl.loop(0, NSL)
                def _r(i):
                    rb[i, :] = 1.0 / (dc[i, :] + EPS)

                pltpu.sync_copy(
                    rb, rden_hbm.at[pl.ds(_al(g * NP + sid * NSL), NSL), :])

    return kb(as1p, ad1p, srcg, dstg)


def _stage_c(h1g, w, rden, srcg, dstg):
    mesh = plsc.VectorSubcoreMesh(core_axis_name="c", subcore_axis_name="s")
    ZR = 160

    @functools.partial(
        pl.kernel,
        out_type=jax.ShapeDtypeStruct((AG, 2, NP, 128), jnp.float32),
        mesh=mesh,
        compiler_params=pltpu.CompilerParams(use_tc_tiling_on_sc=False, needs_layout_passes=False),
        scratch_types=[
            pltpu.VMEM_SHARED((NP, 128), jnp.float32),
            [pltpu.VMEM((CHK,), jnp.int32)] * 2,
            [pltpu.VMEM((CHK,), jnp.int32)] * 2,
            [pltpu.VMEM((CHK,), jnp.int32)] * 2,
            [pltpu.VMEM((CHK,), jnp.int32)] * 2,
            [pltpu.VMEM((CHK, 128), jnp.float32)] * 2,
            [pltpu.VMEM((CHK, LL), jnp.float32)] * 2,
            [pltpu.VMEM((CHK, LL), jnp.float32)] * 2,
            pltpu.VMEM((ZR, 128), jnp.float32),
            [pltpu.SemaphoreType.DMA] * 2,
            [pltpu.SemaphoreType.DMA] * 2,
            [pltpu.SemaphoreType.DMA] * 2,
            [pltpu.SemaphoreType.DMA] * 2,
        ],
    )
    def kc(h_hbm, w_hbm, rd_hbm, src_hbm, dst_hbm, out_hbm,
           acc_sp, sv, dv, dl, gi, hr, wr, rr, zb, semh, semr, semw, sems):
        core = lax.axis_index("c")
        sid = lax.axis_index("s")

        @pl.loop(0, ZR)
        def _zb(i):
            for v in range(8):
                zb[i, pl.ds(v * LL, LL)] = jnp.zeros((LL,), jnp.float32)

        for g in range(AG):
            for grp in range(2):
                @pl.when(core == grp)
                def _t():
                    for j in range(NSL // ZR):
                        pltpu.sync_copy(
                            zb, acc_sp.at[pl.ds(_al(sid * NSL + j * ZR), ZR)])
                    plsc.subcore_barrier()

                    def wait_sct(b):
                        pltpu.make_async_copy(
                            hr[b], acc_sp.at[dl[b]], sems[b]).wait()

                    def start(ci, b):
                        base = _al(g * EE + sid * EPT + ci * CHK)
                        pltpu.sync_copy(src_hbm.at[pl.ds(base, CHK)], sv[b])
                        pltpu.sync_copy(dst_hbm.at[pl.ds(base, CHK)], dv[b])
                        for k in range(CHK // LL):
                            s16 = sv[b][pl.ds(k * LL, LL)]
                            gi[b][pl.ds(k * LL, LL)] = s16 * 2 + grp
                            d16 = dv[b][pl.ds(k * LL, LL)]
                            dl[b][pl.ds(k * LL, LL)] = d16 - g * NP
                        pltpu.async_copy(h_hbm.at[gi[b]], hr[b], semh[b])
                        pltpu.async_copy(rd_hbm.at[dv[b]], rr[b], semr[b])
                        pltpu.async_copy(
                            w_hbm.at[pl.ds(base, CHK), :], wr[b], semw[b])

                    def process(b):
                        pltpu.make_async_copy(
                            h_hbm.at[gi[b]], hr[b], semh[b]).wait()
                        pltpu.make_async_copy(
                            rd_hbm.at[dv[b]], rr[b], semr[b]).wait()
                        pltpu.make_async_copy(
                            w_hbm.at[pl.ds(0, CHK), :], wr[b], semw[b]).wait()

                        @pl.loop(0, CHK)
                        def _e(i):
                            arow = wr[b][i, :] * rr[b][i, :]
                            for j in range(4):
                                a = arow[grp * 4 + j]
                                for v in range(2):
                                    sl = pl.ds(j * 32 + v * LL, LL)
                                    hr[b][i, sl] = hr[b][i, sl] * a

                        pltpu.async_copy(
                            hr[b], acc_sp.at[dl[b]], sems[b], add=True)

                    start(0, 0)

                    @pl.loop(0, NCHK // 2)
                    def _pair(i):
                        @pl.when(i > 0)
                        def _w1():
                            wait_sct(1)
                        start(2 * i + 1, 1)
                        process(0)
                        wait_sct(0)
                        start(2 * i + 2, 0)
                        process(1)

                    wait_sct(1)
                    process(0)
                    wait_sct(0)
                    plsc.subcore_barrier()
                    pltpu.sync_copy(
                        acc_sp.at[pl.ds(_al(sid * NSL), NSL)],
                        out_hbm.at[g, grp, pl.ds(_al(sid * NSL), NSL), :])

    return kc(h1g, w, rden, srcg, dstg)


def _stage_d(acc, b1p, W2p):
    def body(a_ref, b_ref, w_ref, o_ref):
        h0 = jnp.maximum(a_ref[0, 0] + b_ref[0], 0.0)
        h1 = jnp.maximum(a_ref[0, 1] + b_ref[1], 0.0)
        o_ref[0] = (jnp.dot(h0, w_ref[0], preferred_element_type=jnp.float32)
                    + jnp.dot(h1, w_ref[1],
                              preferred_element_type=jnp.float32))

    blk = 1024
    return pl.pallas_call(
        body,
        grid=(AG, NP // blk),
        in_specs=[
            pl.BlockSpec((1, 2, blk, 128), lambda g, i: (g, 0, i, 0)),
            pl.BlockSpec((2, 128), lambda g, i: (0, 0)),
            pl.BlockSpec((2, 128, 1), lambda g, i: (0, 0, 0)),
        ],
        out_specs=pl.BlockSpec((1, blk, 1), lambda g, i: (g, i, 0)),
        out_shape=jax.ShapeDtypeStruct((AG, NP, 1), jnp.float32),
    )(acc, b1p, W2p)


def _stage_e(s2, d2, h2, srcg, dstg):
    mesh = plsc.VectorSubcoreMesh(core_axis_name="c", subcore_axis_name="s")
    NR = NP // LL
    RS = NR // NS

    @functools.partial(
        pl.kernel,
        out_type=[
            jax.ShapeDtypeStruct((AG, NR, LL), jnp.float32),
            jax.ShapeDtypeStruct((AG, NR, LL), jnp.float32),
            jax.ShapeDtypeStruct((AG, NP), jnp.float32),
        ],
        mesh=mesh,
        compiler_params=pltpu.CompilerParams(use_tc_tiling_on_sc=False, needs_layout_passes=False),
        scratch_types=[
            pltpu.VMEM_SHARED((NR, LL), jnp.float32),
            pltpu.VMEM_SHARED((NR, LL), jnp.float32),
            pltpu.VMEM_SHARED((NR, LL), jnp.float32),
            pltpu.VMEM((NP,), jnp.float32),
            pltpu.VMEM((NP,), jnp.float32),
            pltpu.VMEM((NP,), jnp.float32),
            pltpu.VMEM((NR, LL), jnp.float32),
            pltpu.VMEM((NR, LL), jnp.float32),
            pltpu.VMEM((NR, LL), jnp.float32),
            pltpu.VMEM((NR, LL), jnp.float32),
            pltpu.VMEM((NR,), jnp.int32),
            pltpu.VMEM((CHK,), jnp.int32),
            pltpu.VMEM((CHK,), jnp.int32),
            pltpu.VMEM((NP,), jnp.float32),
            pltpu.VMEM((RS, LL), jnp.float32),
            pltpu.VMEM((RS, LL), jnp.float32),
            pltpu.VMEM((NSL,), jnp.float32),
            pltpu.VMEM((NSL,), jnp.int32),
            pltpu.VMEM((RS, LL), jnp.float32),
            pltpu.VMEM((RS, LL), jnp.float32),
        ],
    )
    def ke(s2_hbm, d2_hbm, h2_hbm, src_hbm, dst_hbm,
           out2_hbm, att_hbm, ee2_hbm,
           den_sp, num_sp, cnt_sp,
           s2t, d2t, h2t, denp, nump, cntp, denf, idn,
           sv, dv, eef, nsl, csl, esl, dnv, ob, ab):
        core = lax.axis_index("c")
        sid = lax.axis_index("s")

        @pl.loop(0, RS)
        def _idn(k):
            iota = lax.broadcasted_iota(jnp.int32, (LL,), 0)
            idn[pl.ds(k * LL, LL)] = iota + k * LL

        for g in range(AG):
            @pl.when(core == g % 2)
            def _g():
                @pl.loop(0, NR)
                def _z(i):
                    z = jnp.zeros((LL,), jnp.float32)
                    denp[i, :] = z
                    nump[i, :] = z
                    cntp[i, :] = z

                sl = pl.ds(_al(sid * RS), RS)
                pltpu.sync_copy(denp.at[pl.ds(0, RS)], den_sp.at[sl])
                pltpu.sync_copy(denp.at[pl.ds(0, RS)], num_sp.at[sl])
                pltpu.sync_copy(denp.at[pl.ds(0, RS)], cnt_sp.at[sl])
                pltpu.sync_copy(s2_hbm.at[g], s2t)
                pltpu.sync_copy(d2_hbm.at[g], d2t)
                pltpu.sync_copy(h2_hbm.at[g], h2t)
                plsc.subcore_barrier()

                @pl.loop(0, NCHK)
                def _chunk(ci):
                    base = _al(g * EE + sid * EPT + ci * CHK)
                    pltpu.sync_copy(src_hbm.at[pl.ds(base, CHK)], sv)
                    pltpu.sync_copy(dst_hbm.at[pl.ds(base, CHK)], dv)
                    ones = jnp.ones((LL,), jnp.float32)
                    for k in range(CHK // LL):
                        s16 = sv[pl.ds(k * LL, LL)] - g * NP
                        d16 = dv[pl.ds(k * LL, LL)] - g * NP
                        t = (plsc.load_gather(s2t, [s16])
                             + plsc.load_gather(d2t, [d16]))
                        w2 = jnp.exp(jnp.maximum(t, 0.2 * t))
                        eef[pl.ds(ci * CHK + k * LL, LL)] = w2
                        h2v = plsc.load_gather(h2t, [s16])
                        row = lax.shift_right_logical(d16, 4)
                        col = jnp.bitwise_and(d16, LL - 1)
                        plsc.addupdate_scatter(denp, [row, col], w2)
                        plsc.addupdate_scatter(nump, [row, col], w2 * h2v)
                        plsc.addupdate_scatter(cntp, [row, col], ones)

                @pl.when(sid == 0)
                def _ee():
                    pltpu.sync_copy(eef.at[pl.ds(0, EPT)],
                                    ee2_hbm.at[g, pl.ds(0, EPT)])

                pltpu.sync_copy(denp, den_sp.at[idn], add=True)
                pltpu.sync_copy(nump, num_sp.at[idn], add=True)
                pltpu.sync_copy(cntp, cnt_sp.at[idn], add=True)
                plsc.subcore_barrier()

                pltpu.sync_copy(den_sp, denf)
                pltpu.sync_copy(num_sp.at[sl], nsl)
                pltpu.sync_copy(cnt_sp.at[sl], csl)
                pltpu.sync_copy(ee2_hbm.at[g, pl.ds(_al(sid * NSL), NSL)],
                                esl)
                pltpu.sync_copy(
                    dst_hbm.at[pl.ds(_al(g * EE + sid * NSL), NSL)], dnv)

                @pl.loop(0, RS)
                def _o(r):
                    dr = denf[sid * RS + r, :]
                    ob[r, :] = nsl[r, :] / (dr + EPS)

                pltpu.sync_copy(ob, out2_hbm.at[g, sl, :])

                @pl.loop(0, RS)
                def _a(k):
                    d16 = dnv[pl.ds(k * LL, LL)] - g * NP
                    row = lax.shift_right_logical(d16, 4)
                    col = jnp.bitwise_and(d16, LL - 1)
                    denv = plsc.load_gather(denf, [row, col])
                    e16 = esl[pl.ds(k * LL, LL)]
                    ab[k, :] = csl[k, :] * e16 / (denv + EPS)

                pltpu.sync_copy(ab, att_hbm.at[g, sl, :])

    return ke(s2, d2, h2, srcg, dstg)


def _stage_f(out2, b2p):
    def body(x_ref, b_ref, o_ref):
        s = jnp.sum(x_ref[...]) * (1.0 / NN) + b_ref[0, 0]
        o_ref[...] = jnp.full((1, 1, 128), s, jnp.float32)

    return pl.pallas_call(
        body,
        grid=(AG,),
        in_specs=[
            pl.BlockSpec((1, NP // 128, 128), lambda g: (g, 0, 0)),
            pl.BlockSpec((1, 1), lambda g: (0, 0)),
        ],
        out_specs=pl.BlockSpec((1, 1, 128), lambda g: (g, 0, 0)),
        out_shape=jax.ShapeDtypeStruct((AG, 1, 128), jnp.float32),
    )(out2, b2p)


def kernel(x, edge_index, W1, a_src1, a_dst1, b1, W2, a_src2, a_dst2, b2):
    eye = jnp.eye(NH, dtype=jnp.float32)
    Asp = jnp.pad((a_src1[:, :, None] * eye[:, None, :]).reshape(DD, NH),
                  ((0, 0), (0, NH)))
    Adp = jnp.pad((a_dst1[:, :, None] * eye[:, None, :]).reshape(DD, NH),
                  ((0, 0), (0, NH)))
    off = (jnp.arange(AG, dtype=jnp.int32) * NP)[:, None]
    srcg = (edge_index[:, 0, :] + off).reshape(AG * EE)
    dstg = (edge_index[:, 1, :] + off).reshape(AG * EE)

    h1, as1p, ad1p = _stage_a(x, W1, Asp, Adp)
    padn = ((0, 0), (0, NP - NN), (0, 0))
    as1p = jnp.pad(as1p, padn).reshape(AG * NP, LL)
    ad1p = jnp.pad(ad1p, padn).reshape(AG * NP, LL)
    h1g = jnp.pad(h1, padn).reshape(AG * NP * 2, 128)
    w, rden = _stage_b(as1p, ad1p, srcg, dstg)
    acc = _stage_c(h1g, w, rden, srcg, dstg)
    h2 = _stage_d(acc, b1.reshape(2, 128), W2.reshape(2, 128, 1))
    h2f = h2[:, :, 0]
    s2 = h2f * a_src2[0, 0]
    d2 = h2f * a_dst2[0, 0]
    out2, att, _ = _stage_e(s2, d2, h2f, srcg, dstg)
    probas = _stage_f(out2.reshape(AG, NP // 128, 128), b2.reshape(1, 1))
    proba_vec = probas[:, 0, 0]
    best = jnp.argmax(proba_vec)
    attention = jnp.take(att.reshape(AG, NP)[:, :NN], best, axis=0)
    return proba_vec, attention
